# baseline (speedup 1.0000x reference)
import functools

import jax
import jax.numpy as jnp
from jax.experimental import pallas as pl
from jax.experimental.pallas import tpu as pltpu

_EOS = 2
_NEG_INF = float("-inf")


def _mask_kernel(scores_ref, m8_ref, out_ref, any_ref, *, nv):
    i = pl.program_id(0)
    m = m8_ref[...] != 0
    out_ref[...] = jnp.where(m, scores_ref[...], _NEG_INF)
    local = jnp.max(m.astype(jnp.int32), axis=1, keepdims=True)
    prev = jnp.where(i == 0, jnp.zeros_like(local), any_ref[...])
    any_ref[...] = jnp.maximum(prev, local)

    @pl.when(i == nv - 1)
    def _eos_fixup():
        acc = any_ref[...]
        s = scores_ref[:, :128]
        mm = m8_ref[:, :128] != 0
        col = jax.lax.broadcasted_iota(jnp.int32, (1, 128), 1)
        force = jnp.logical_and(col == _EOS, acc == 0)
        out_ref[:, :128] = jnp.where(jnp.logical_or(mm, force), s, _NEG_INF)


def kernel(input_ids, scores, allowed_mask):
    del input_ids
    b, v = scores.shape
    vb = 25088
    nv = pl.cdiv(v, vb)
    vpad = nv * vb
    m8 = jnp.pad(allowed_mask.astype(jnp.int8), ((0, 0), (0, vpad - v)))
    idx = lambda vi: (0, jax.lax.rem(vi + 1, nv))
    return pl.pallas_call(
        functools.partial(_mask_kernel, nv=nv),
        grid=(nv,),
        in_specs=[
            pl.BlockSpec((b, vb), idx),
            pl.BlockSpec((b, vb), idx),
        ],
        out_specs=pl.BlockSpec((b, vb), idx),
        out_shape=jax.ShapeDtypeStruct((b, v), scores.dtype),
        scratch_shapes=[pltpu.VMEM((b, 1), jnp.int32)],
    )(scores, m8)
